# Initial kernel scaffold; baseline (speedup 1.0000x reference)
import jax
import jax.numpy as jnp
from jax.experimental import pallas as pl


def kernel(points, features, neighbor_indices):
    raise NotImplementedError("write your pallas kernel here")


# speedup vs baseline: 2.5272x; 2.5272x over previous
import functools

import jax
import jax.numpy as jnp
from jax import lax
from jax.experimental import pallas as pl
from jax.experimental.pallas import tpu as pltpu
from jax.experimental.pallas import tpu_sc as plsc

N = 10000
F = 256
K = 16
PTS_PER_UNIT = 8
IDX_PER_UNIT = PTS_PER_UNIT * K
NUM_UNITS = N // PTS_PER_UNIT
LANES = 16
COLS = F // LANES

_info = plsc.get_sparse_core_info()
NC, NS = _info.num_cores, _info.num_subcores
NW = NC * NS


def _pool_kernel(feat_hbm, idx_hbm, out_hbm, idx_v, rows_v, out_v, sem):
    wid = lax.axis_index("s") * NC + lax.axis_index("c")
    n_units = (NUM_UNITS - wid + NW - 1) // NW

    def unit_body(i, carry):
        u = wid + i * NW
        pltpu.sync_copy(idx_hbm.at[pl.ds(u * IDX_PER_UNIT, IDX_PER_UNIT)], idx_v)
        pltpu.async_copy(feat_hbm.at[idx_v], rows_v, sem).wait()

        def point_body(p, carry2):
            base = p * K
            accs = tuple(rows_v[base, pl.ds(c * LANES, LANES)] for c in range(COLS))

            def row_body(r, accs):
                return tuple(
                    jnp.maximum(a, rows_v[base + r, pl.ds(c * LANES, LANES)])
                    for c, a in enumerate(accs)
                )

            accs = lax.fori_loop(1, K, row_body, accs)
            for c in range(COLS):
                out_v[p, pl.ds(c * LANES, LANES)] = accs[c]
            return carry2

        lax.fori_loop(0, PTS_PER_UNIT, point_body, 0)
        pltpu.sync_copy(out_v, out_hbm.at[pl.ds(u * PTS_PER_UNIT, PTS_PER_UNIT)])
        return carry

    lax.fori_loop(0, n_units, unit_body, 0)


@jax.jit
def _pool(features, idx_flat):
    mesh = plsc.VectorSubcoreMesh(core_axis_name="c", subcore_axis_name="s")
    run = functools.partial(
        pl.kernel,
        mesh=mesh,
        out_type=jax.ShapeDtypeStruct((N, F), jnp.float32),
        scratch_types=[
            pltpu.VMEM((IDX_PER_UNIT,), jnp.int32),
            pltpu.VMEM((IDX_PER_UNIT, F), jnp.float32),
            pltpu.VMEM((PTS_PER_UNIT, F), jnp.float32),
            pltpu.SemaphoreType.DMA,
        ],
    )(_pool_kernel)
    return run(features, idx_flat)


def kernel(points, features, neighbor_indices):
    del points
    idx_flat = neighbor_indices.astype(jnp.int32).reshape(-1)
    return _pool(features, idx_flat)
